# baseline (speedup 1.0000x reference)
import functools

import numpy as np
import jax
import jax.numpy as jnp
from jax import lax
from jax.experimental import pallas as pl
from jax.experimental.pallas import tpu as pltpu
from jax.experimental.pallas import tpu_sc as plsc

D_MODEL = 768
MAX_LEN = 5000

NUM_CORES = 2
NUM_SUBCORES = 16
NUM_WORKERS = NUM_CORES * NUM_SUBCORES
LANES = 16


def _pe_table(time_steps: int) -> np.ndarray:
    half_dim = D_MODEL // 2
    pe = np.zeros((D_MODEL, MAX_LEN), dtype=np.float64)
    pos = np.arange(MAX_LEN)
    freq = 10000 ** (2 * np.arange(half_dim) / D_MODEL)
    pos_freq = pos.reshape((1, -1)) / freq.reshape((-1, 1))
    pe[:half_dim, :] = np.sin(pos_freq)
    pe[half_dim:, :] = np.cos(pos_freq)
    return pe.T[:time_steps].astype(np.float32)


@functools.partial(jax.jit, static_argnames=("batch", "seq_len"))
def _sc_embed(x_flat, pe, table, *, batch, seq_len):
    rows_total = batch * seq_len
    chunk = seq_len // NUM_WORKERS
    scale = float(np.sqrt(np.float32(D_MODEL)))
    vregs_per_row = D_MODEL // LANES

    mesh = plsc.VectorSubcoreMesh(
        core_axis_name="c", subcore_axis_name="s")

    @functools.partial(
        pl.kernel,
        out_type=jax.ShapeDtypeStruct((rows_total, D_MODEL), jnp.float32),
        mesh=mesh,
        scratch_types=[
            pltpu.VMEM((chunk,), jnp.int32),
            pltpu.VMEM((chunk, D_MODEL), jnp.float32),
            pltpu.VMEM((chunk, D_MODEL), jnp.float32),
            pltpu.SemaphoreType.DMA,
        ],
    )
    def k(x_hbm, pe_hbm, table_hbm, out_hbm, idx_v, pe_v, rows_v, sem):
        wid = lax.axis_index("s") * NUM_CORES + lax.axis_index("c")
        t0 = wid * chunk
        pltpu.sync_copy(pe_hbm.at[pl.ds(t0, chunk)], pe_v)
        for b in range(batch):
            base = b * seq_len + t0
            pltpu.sync_copy(x_hbm.at[pl.ds(base, chunk)], idx_v)
            pltpu.async_copy(table_hbm.at[idx_v], rows_v, sem).wait()

            def row_body(r, _):
                for c in range(vregs_per_row):
                    sl = pl.ds(c * LANES, LANES)
                    rows_v[r, sl] = rows_v[r, sl] * scale + pe_v[r, sl]
                return 0

            lax.fori_loop(0, chunk, row_body, 0)
            pltpu.sync_copy(rows_v, out_hbm.at[pl.ds(base, chunk)])

    return k(x_flat, pe, table)


def kernel(x, table):
    batch, seq_len = x.shape
    pe = jnp.asarray(_pe_table(seq_len))
    out = _sc_embed(x.reshape(-1), pe, table, batch=batch, seq_len=seq_len)
    return out.reshape(batch, seq_len, D_MODEL)
